# Initial kernel scaffold; baseline (speedup 1.0000x reference)
import functools

import jax
import jax.numpy as jnp
from jax import lax
from jax.experimental import pallas as pl
from jax.experimental.pallas import tpu as pltpu
from jax.experimental.pallas import tpu_sc as plsc

NN = 100000
CC = 16
EE = 3200000
KK = 3
AA = 0.9

NC = 2
NS = 16
NW = NC * NS

B = 80
SCH = 10
EPW = EE // NW
NCHUNK = EPW // B
NSUPER = NCHUNK // SCH

RPT_SC = NN // NS
RPT_W = NN // NW
RCH = 125
NRCH = RPT_W // RCH

_mesh = plsc.VectorSubcoreMesh(
    core_axis_name="c", subcore_axis_name="s", num_cores=NC, num_subcores=NS
)


def _worker_id():
    return lax.axis_index("s") * NC + lax.axis_index("c")


@functools.partial(
    pl.kernel,
    out_type=jax.ShapeDtypeStruct((NC, NN, 2), jnp.float32),
    mesh=_mesh,
    scratch_types=[
        pltpu.VMEM((SCH, B), jnp.int32),
        pltpu.VMEM((SCH, B), jnp.int32),
        pltpu.VMEM((B, 2), jnp.float32),
        pltpu.VMEM((B, 2), jnp.float32),
        pltpu.VMEM_SHARED((NN, 2), jnp.float32),
    ],
)
def _degrees(src2d, dst2d, onesin_hbm, onesout_hbm, zeros2_hbm, out,
             sidx, didx, onein, oneout, acc):
    cid = lax.axis_index("c")
    sid = lax.axis_index("s")
    wid = _worker_id()
    pltpu.sync_copy(onesin_hbm, onein)
    pltpu.sync_copy(onesout_hbm, oneout)
    pltpu.sync_copy(zeros2_hbm, acc.at[pl.ds(sid * RPT_SC, RPT_SC)])
    plsc.subcore_barrier()
    base_rows = wid * NCHUNK

    @pl.loop(0, NSUPER)
    def _(g):
        row0 = base_rows + g * SCH
        pltpu.sync_copy(src2d.at[pl.ds(row0, SCH)], sidx)
        pltpu.sync_copy(dst2d.at[pl.ds(row0, SCH)], didx)
        for s in range(SCH):
            pltpu.sync_copy(onein, acc.at[didx.at[s]], add=True)
            pltpu.sync_copy(oneout, acc.at[sidx.at[s]], add=True)

    plsc.subcore_barrier()
    pltpu.sync_copy(acc.at[pl.ds(sid * RPT_SC, RPT_SC)],
                    out.at[cid, pl.ds(sid * RPT_SC, RPT_SC)])


@functools.partial(
    pl.kernel,
    out_type=jax.ShapeDtypeStruct((NC, NN, CC), jnp.float32),
    mesh=_mesh,
    scratch_types=[
        pltpu.VMEM((SCH, B), jnp.int32),
        pltpu.VMEM((SCH, B), jnp.int32),
        pltpu.VMEM((B, CC), jnp.float32),
        pltpu.VMEM((B, CC), jnp.float32),
        pltpu.VMEM_SHARED((NN, CC), jnp.float32),
        pltpu.SemaphoreType.DMA,
        pltpu.SemaphoreType.DMA,
    ],
)
def _scatter(h_hbm, src2d, dst2d, zeros_hbm, out,
             sidx, didx, rows0, rows1, acc, sem0, sem1):
    cid = lax.axis_index("c")
    sid = lax.axis_index("s")
    wid = _worker_id()
    pltpu.sync_copy(zeros_hbm, acc.at[pl.ds(sid * RPT_SC, RPT_SC)])
    plsc.subcore_barrier()
    base_rows = wid * NCHUNK
    rows = (rows0, rows1)
    sems = (sem0, sem1)

    @pl.loop(0, NSUPER)
    def _(g):
        row0 = base_rows + g * SCH
        pltpu.sync_copy(src2d.at[pl.ds(row0, SCH)], sidx)
        pltpu.sync_copy(dst2d.at[pl.ds(row0, SCH)], didx)
        pend = [None, None]
        pend[0] = pltpu.async_copy(h_hbm.at[sidx.at[0]], rows[0], sems[0])
        for s in range(SCH):
            if s + 1 < SCH:
                j = (s + 1) % 2
                pend[j] = pltpu.async_copy(h_hbm.at[sidx.at[s + 1]], rows[j], sems[j])
            pend[s % 2].wait()
            pltpu.sync_copy(rows[s % 2], acc.at[didx.at[s]], add=True)

    plsc.subcore_barrier()
    pltpu.sync_copy(acc.at[pl.ds(sid * RPT_SC, RPT_SC)],
                    out.at[cid, pl.ds(sid * RPT_SC, RPT_SC)])


@functools.partial(
    pl.kernel,
    out_type=(jax.ShapeDtypeStruct((NN, CC), jnp.float32),
              jax.ShapeDtypeStruct((NN, CC), jnp.float32)),
    mesh=_mesh,
    scratch_types=[pltpu.VMEM((RCH, CC), jnp.float32)] * 7,
)
def _update(part, init_hbm, ni_hbm, nj_hbm, y_out, h_out,
            p0, p1, ini, ni, nj, yb, hb):
    wid = _worker_id()
    base = wid * RPT_W

    @pl.loop(0, NRCH)
    def _(t):
        r0 = base + t * RCH
        pltpu.sync_copy(part.at[0, pl.ds(r0, RCH)], p0)
        pltpu.sync_copy(part.at[1, pl.ds(r0, RCH)], p1)
        pltpu.sync_copy(init_hbm.at[pl.ds(r0, RCH)], ini)
        pltpu.sync_copy(ni_hbm.at[pl.ds(r0, RCH)], ni)
        pltpu.sync_copy(nj_hbm.at[pl.ds(r0, RCH)], nj)

        @pl.loop(0, RCH)
        def _(r):
            agg = p0[r, :] + p1[r, :]
            y = ini[r, :] + jnp.float32(AA) * agg * ni[r, :]
            y = jnp.minimum(jnp.maximum(y, jnp.float32(0.0)), jnp.float32(1.0))
            yb[r, :] = y
            hb[r, :] = y * nj[r, :]

        pltpu.sync_copy(yb, y_out.at[pl.ds(r0, RCH)])
        pltpu.sync_copy(hb, h_out.at[pl.ds(r0, RCH)])


_BT = 2000


def _prep_body(lab_ref, msk_ref, deg_ref, init_ref, h0_ref, ni_ref, nj_ref):
    lab = lab_ref[...]
    msk = msk_ref[...]
    deg = deg_ref[0] + deg_ref[1]
    nrm = lax.rsqrt(jnp.maximum(deg, jnp.float32(1.0)))
    ni = jnp.broadcast_to(nrm[:, 0:1], (_BT, CC))
    nj = jnp.broadcast_to(nrm[:, 1:2], (_BT, CC))
    y0 = jnp.where(msk > 0, lab, jnp.float32(0.0))
    init_ref[...] = jnp.float32(1.0 - AA) * y0
    ni_ref[...] = ni
    nj_ref[...] = nj
    h0_ref[...] = y0 * nj


def _prep(labels, mask_i, degparts):
    grid = (NN // _BT,)
    fspec = pl.BlockSpec((_BT, CC), lambda i: (i, 0))
    return pl.pallas_call(
        _prep_body,
        grid=grid,
        in_specs=[
            fspec,
            pl.BlockSpec((_BT, 1), lambda i: (i, 0)),
            pl.BlockSpec((2, _BT, 2), lambda i: (0, i, 0)),
        ],
        out_specs=[fspec, fspec, fspec, fspec],
        out_shape=[jax.ShapeDtypeStruct((NN, CC), jnp.float32)] * 4,
    )(labels, mask_i, degparts)


def kernel(labels, mask, edge_index):
    labels = labels.astype(jnp.float32)
    src2d = edge_index[0].reshape(EE // B, B)
    dst2d = edge_index[1].reshape(EE // B, B)
    mask_i = mask.astype(jnp.int32).reshape(NN, 1)

    onesin = jnp.broadcast_to(jnp.array([1.0, 0.0], jnp.float32), (B, 2))
    onesout = jnp.broadcast_to(jnp.array([0.0, 1.0], jnp.float32), (B, 2))
    zeros2 = jnp.zeros((RPT_SC, 2), jnp.float32)
    zerosC = jnp.zeros((RPT_SC, CC), jnp.float32)

    degparts = _degrees(src2d, dst2d, onesin, onesout, zeros2)
    init, h, ni, nj = _prep(labels, mask_i, degparts)
    y = None
    for _ in range(KK):
        part = _scatter(h, src2d, dst2d, zerosC)
        y, h = _update(part, init, ni, nj)
    return y


# speedup vs baseline: 18.7255x; 18.7255x over previous
import functools

import jax
import jax.numpy as jnp
from jax import lax
from jax.experimental import pallas as pl
from jax.experimental.pallas import tpu as pltpu
from jax.experimental.pallas import tpu_sc as plsc

NN = 100000
NP = 102400
CC = 16
EE = 3200000
KK = 3
AA = 0.9

NC = 2
NS = 16
NW = NC * NS

B = 80
SCH = 10
EPW = EE // NW
NCHUNK = EPW // B
NSUPER = NCHUNK // SCH

RPT_SC = NP // NS
RPT_W = NP // NW
RCH = 128
CHW = RCH * CC
NRCH = RPT_W // RCH

_mesh = plsc.VectorSubcoreMesh(
    core_axis_name="c", subcore_axis_name="s", num_cores=NC, num_subcores=NS
)

_f32 = jnp.float32
_i32 = jnp.int32


def _ids():
    cid = lax.axis_index("c")
    sid = lax.axis_index("s")
    return cid, sid, sid * NC + cid


@functools.partial(
    pl.kernel,
    out_type=jax.ShapeDtypeStruct((2 * NP, CC), _f32),
    mesh=_mesh,
    compiler_params=pltpu.CompilerParams(use_tc_tiling_on_sc=False),
    scratch_types=(
        [pltpu.VMEM((B,), _i32) for _ in range(2 * SCH)]
        + [pltpu.VMEM((B, CC), _f32), pltpu.VMEM((B, CC), _f32),
           pltpu.VMEM_SHARED((NP, CC), _f32),
           pltpu.SemaphoreType.DMA]
    ),
)
def _degrees(src5, dst5, onesin_hbm, onesout_hbm, zerosC_hbm, out, *scr):
    sslot = scr[:SCH]
    dslot = scr[SCH:2 * SCH]
    onein, oneout, acc, isem = scr[2 * SCH:]
    cid, sid, wid = _ids()
    pltpu.sync_copy(onesin_hbm, onein)
    pltpu.sync_copy(onesout_hbm, oneout)
    pltpu.sync_copy(zerosC_hbm, acc.at[pl.ds(sid * RPT_SC, RPT_SC)])
    plsc.subcore_barrier()

    @pl.loop(0, NSUPER)
    def _(g):
        pend = []
        for s in range(SCH):
            pend.append(pltpu.async_copy(src5.at[wid, g, s, 0], sslot[s], isem))
            pend.append(pltpu.async_copy(dst5.at[wid, g, s, 0], dslot[s], isem))
        for p in pend:
            p.wait()
        for s in range(SCH):
            pltpu.sync_copy(onein, acc.at[dslot[s]], add=True)
            pltpu.sync_copy(oneout, acc.at[sslot[s]], add=True)

    plsc.subcore_barrier()
    sl = pl.ds(sid * RPT_SC, RPT_SC)
    pltpu.sync_copy(acc.at[sl], out.at[pl.ds(cid * NP + sid * RPT_SC, RPT_SC)])


@functools.partial(
    pl.kernel,
    out_type=jax.ShapeDtypeStruct((2 * NP, CC), _f32),
    mesh=_mesh,
    compiler_params=pltpu.CompilerParams(use_tc_tiling_on_sc=False),
    scratch_types=(
        [pltpu.VMEM((B,), _i32) for _ in range(2 * SCH)]
        + [pltpu.VMEM((B, CC), _f32), pltpu.VMEM((B, CC), _f32),
           pltpu.VMEM_SHARED((NP, CC), _f32),
           pltpu.SemaphoreType.DMA,
           pltpu.SemaphoreType.DMA,
           pltpu.SemaphoreType.DMA]
    ),
)
def _scatter(h_hbm, src5, dst5, zeros_hbm, out, *scr):
    sslot = scr[:SCH]
    dslot = scr[SCH:2 * SCH]
    rows = scr[2 * SCH:2 * SCH + 2]
    acc = scr[2 * SCH + 2]
    isem, gsem0, gsem1 = scr[2 * SCH + 3:]
    gsems = (gsem0, gsem1)
    cid, sid, wid = _ids()
    pltpu.sync_copy(zeros_hbm, acc.at[pl.ds(sid * RPT_SC, RPT_SC)])
    plsc.subcore_barrier()

    @pl.loop(0, NSUPER)
    def _(g):
        pend = []
        for s in range(SCH):
            pend.append(pltpu.async_copy(src5.at[wid, g, s, 0], sslot[s], isem))
            pend.append(pltpu.async_copy(dst5.at[wid, g, s, 0], dslot[s], isem))
        for p in pend:
            p.wait()
        gp = [None, None]
        gp[0] = pltpu.async_copy(h_hbm.at[sslot[0]], rows[0], gsems[0])
        for s in range(SCH):
            if s + 1 < SCH:
                j = (s + 1) % 2
                gp[j] = pltpu.async_copy(h_hbm.at[sslot[s + 1]], rows[j], gsems[j])
            gp[s % 2].wait()
            pltpu.sync_copy(rows[s % 2], acc.at[dslot[s]], add=True)

    plsc.subcore_barrier()
    sl = pl.ds(sid * RPT_SC, RPT_SC)
    pltpu.sync_copy(acc.at[sl], out.at[pl.ds(cid * NP + sid * RPT_SC, RPT_SC)])


@functools.partial(
    pl.kernel,
    out_type=(jax.ShapeDtypeStruct((NP * CC,), _f32),
              jax.ShapeDtypeStruct((NP * CC,), _f32)),
    mesh=_mesh,
    compiler_params=pltpu.CompilerParams(use_tc_tiling_on_sc=False),
    scratch_types=[pltpu.VMEM((CHW,), _f32)] * 7,
)
def _update(part_hbm, init_hbm, ni_hbm, nj_hbm, y_out, h_out,
            p0, p1, ini, ni, nj, yb, hb):
    _, _, wid = _ids()
    base = wid * RPT_W * CC

    @pl.loop(0, NRCH)
    def _(t):
        e0 = base + t * CHW
        sl = pl.ds(e0, CHW)
        pltpu.sync_copy(part_hbm.at[sl], p0)
        pltpu.sync_copy(part_hbm.at[pl.ds(NP * CC + e0, CHW)], p1)
        pltpu.sync_copy(init_hbm.at[sl], ini)
        pltpu.sync_copy(ni_hbm.at[sl], ni)
        pltpu.sync_copy(nj_hbm.at[sl], nj)

        @pl.loop(0, RCH, unroll=4)
        def _(r):
            v = pl.ds(r * CC, CC)
            agg = p0[v] + p1[v]
            y = ini[v] + _f32(AA) * agg * ni[v]
            y = jnp.minimum(jnp.maximum(y, _f32(0.0)), _f32(1.0))
            yb[v] = y
            hb[v] = y * nj[v]

        pltpu.sync_copy(yb, y_out.at[sl])
        pltpu.sync_copy(hb, h_out.at[sl])


_BT = 2048


def _prep_body(lab_ref, msk_ref, dga_ref, dgb_ref,
               init_ref, h0_ref, ni_ref, nj_ref):
    lab = lab_ref[...]
    msk = msk_ref[...]
    deg = dga_ref[...] + dgb_ref[...]
    nrm = lax.rsqrt(jnp.maximum(deg, _f32(1.0)))
    ni = jnp.broadcast_to(nrm[:, 0:1], (_BT, CC))
    nj = jnp.broadcast_to(nrm[:, 8:9], (_BT, CC))
    y0 = jnp.where(msk > 0, lab, _f32(0.0))
    init_ref[...] = _f32(1.0 - AA) * y0
    ni_ref[...] = ni
    nj_ref[...] = nj
    h0_ref[...] = y0 * nj


def _prep(labels_p, mask_p, deg2):
    fspec = pl.BlockSpec((_BT, CC), lambda i: (i, 0))
    dspec_a = pl.BlockSpec((_BT, CC), lambda i: (i, 0))
    dspec_b = pl.BlockSpec((_BT, CC), lambda i: (i + NP // _BT, 0))
    return pl.pallas_call(
        _prep_body,
        grid=(NP // _BT,),
        in_specs=[fspec, pl.BlockSpec((_BT, 1), lambda i: (i, 0)),
                  dspec_a, dspec_b],
        out_specs=[fspec, fspec, fspec, fspec],
        out_shape=[jax.ShapeDtypeStruct((NP, CC), _f32)] * 4,
    )(labels_p, mask_p, deg2, deg2)


def kernel(labels, mask, edge_index):
    labels = labels.astype(_f32)
    src5 = edge_index[0].reshape(NW, NSUPER, SCH, 1, B)
    dst5 = edge_index[1].reshape(NW, NSUPER, SCH, 1, B)

    labels_p = jnp.pad(labels, ((0, NP - NN), (0, 0)))
    mask_p = jnp.pad(mask.astype(_i32), (0, NP - NN)).reshape(NP, 1)

    onesin = jnp.broadcast_to(
        jnp.array([1.0] * 8 + [0.0] * 8, _f32), (B, CC))
    onesout = jnp.broadcast_to(
        jnp.array([0.0] * 8 + [1.0] * 8, _f32), (B, CC))
    zerosC = jnp.zeros((RPT_SC, CC), _f32)

    deg2 = _degrees(src5, dst5, onesin, onesout, zerosC)
    init, h, ni, nj = _prep(labels_p, mask_p, deg2)
    init_f = init.reshape(NP * CC)
    ni_f = ni.reshape(NP * CC)
    nj_f = nj.reshape(NP * CC)

    y_f = None
    for _ in range(KK):
        part = _scatter(h, src5, dst5, zerosC)
        y_f, h_f = _update(part.reshape(2 * NP * CC), init_f, ni_f, nj_f)
        h = h_f.reshape(NP, CC)
    return y_f.reshape(NP, CC)[:NN]


# speedup vs baseline: 19.7609x; 1.0553x over previous
import functools

import jax
import jax.numpy as jnp
from jax import lax
from jax.experimental import pallas as pl
from jax.experimental.pallas import tpu as pltpu
from jax.experimental.pallas import tpu_sc as plsc

NN = 100000
NP = 102400
CC = 16
EE = 3200000
KK = 3
AA = 0.9

NC = 2
NS = 16
NW = NC * NS

B = 80
SCH = 10
EPW = EE // NW
NCHUNK = EPW // B
NSUPER = NCHUNK // SCH

RPT_SC = NP // NS
RPT_W = NP // NW
RCH = 128
CHW = RCH * CC
NRCH = RPT_W // RCH

_mesh = plsc.VectorSubcoreMesh(
    core_axis_name="c", subcore_axis_name="s", num_cores=NC, num_subcores=NS
)

_f32 = jnp.float32
_i32 = jnp.int32


def _ids():
    cid = lax.axis_index("c")
    sid = lax.axis_index("s")
    return cid, sid, sid * NC + cid


@functools.partial(
    pl.kernel,
    out_type=jax.ShapeDtypeStruct((2 * NP, CC), _f32),
    mesh=_mesh,
    compiler_params=pltpu.CompilerParams(use_tc_tiling_on_sc=False),
    scratch_types=(
        [pltpu.VMEM((B,), _i32) for _ in range(2 * SCH)]
        + [pltpu.VMEM((B, CC), _f32), pltpu.VMEM((B, CC), _f32),
           pltpu.VMEM_SHARED((NP, CC), _f32),
           pltpu.SemaphoreType.DMA,
           pltpu.SemaphoreType.DMA,
           pltpu.SemaphoreType.DMA]
    ),
)
def _degrees(src5, dst5, onesin_hbm, onesout_hbm, zerosC_hbm, out, *scr):
    sslot = scr[:SCH]
    dslot = scr[SCH:2 * SCH]
    onein, oneout, acc, isem, dsem, ssem = scr[2 * SCH:]
    cid, sid, wid = _ids()
    pltpu.sync_copy(onesin_hbm, onein)
    pltpu.sync_copy(onesout_hbm, oneout)
    pltpu.sync_copy(zerosC_hbm, acc.at[pl.ds(sid * RPT_SC, RPT_SC)])
    plsc.subcore_barrier()

    @pl.loop(0, NSUPER)
    def _(g):
        pend = []
        for s in range(SCH):
            pend.append(pltpu.async_copy(src5.at[wid, g, s, 0], sslot[s], isem))
            pend.append(pltpu.async_copy(dst5.at[wid, g, s, 0], dslot[s], isem))
        for p in pend:
            p.wait()
        dp = []
        for s in range(SCH):
            dp.append(pltpu.async_copy(onein, acc.at[dslot[s]], dsem, add=True))
            dp.append(pltpu.async_copy(oneout, acc.at[sslot[s]], ssem, add=True))
        for p in dp:
            p.wait()

    plsc.subcore_barrier()
    sl = pl.ds(sid * RPT_SC, RPT_SC)
    pltpu.sync_copy(acc.at[sl], out.at[pl.ds(cid * NP + sid * RPT_SC, RPT_SC)])


@functools.partial(
    pl.kernel,
    out_type=jax.ShapeDtypeStruct((2 * NP, CC), _f32),
    mesh=_mesh,
    compiler_params=pltpu.CompilerParams(use_tc_tiling_on_sc=False),
    scratch_types=(
        [pltpu.VMEM((B,), _i32) for _ in range(2 * SCH)]
        + [pltpu.VMEM((B, CC), _f32), pltpu.VMEM((B, CC), _f32),
           pltpu.VMEM_SHARED((NP, CC), _f32),
           pltpu.SemaphoreType.DMA,
           pltpu.SemaphoreType.DMA,
           pltpu.SemaphoreType.DMA,
           pltpu.SemaphoreType.DMA,
           pltpu.SemaphoreType.DMA]
    ),
)
def _scatter(h_hbm, src5, dst5, zeros_hbm, out, *scr):
    sslot = scr[:SCH]
    dslot = scr[SCH:2 * SCH]
    rows = scr[2 * SCH:2 * SCH + 2]
    acc = scr[2 * SCH + 2]
    isem, gsem0, gsem1, ssem0, ssem1 = scr[2 * SCH + 3:]
    gsems = (gsem0, gsem1)
    ssems = (ssem0, ssem1)
    cid, sid, wid = _ids()
    pltpu.sync_copy(zeros_hbm, acc.at[pl.ds(sid * RPT_SC, RPT_SC)])
    plsc.subcore_barrier()

    @pl.loop(0, NSUPER)
    def _(g):
        pend = []
        for s in range(SCH):
            pend.append(pltpu.async_copy(src5.at[wid, g, s, 0], sslot[s], isem))
            pend.append(pltpu.async_copy(dst5.at[wid, g, s, 0], dslot[s], isem))
        for p in pend:
            p.wait()
        gp = [None, None]
        sp = [None, None]
        gp[0] = pltpu.async_copy(h_hbm.at[sslot[0]], rows[0], gsems[0])
        for s in range(SCH):
            if s + 1 < SCH:
                j = (s + 1) % 2
                if sp[j] is not None:
                    sp[j].wait()
                    sp[j] = None
                gp[j] = pltpu.async_copy(h_hbm.at[sslot[s + 1]], rows[j], gsems[j])
            gp[s % 2].wait()
            sp[s % 2] = pltpu.async_copy(rows[s % 2], acc.at[dslot[s]],
                                         ssems[s % 2], add=True)
        for p in sp:
            if p is not None:
                p.wait()

    plsc.subcore_barrier()
    sl = pl.ds(sid * RPT_SC, RPT_SC)
    pltpu.sync_copy(acc.at[sl], out.at[pl.ds(cid * NP + sid * RPT_SC, RPT_SC)])


@functools.partial(
    pl.kernel,
    out_type=(jax.ShapeDtypeStruct((NP * CC,), _f32),
              jax.ShapeDtypeStruct((NP * CC,), _f32)),
    mesh=_mesh,
    compiler_params=pltpu.CompilerParams(use_tc_tiling_on_sc=False),
    scratch_types=[pltpu.VMEM((CHW,), _f32)] * 7,
)
def _update(part_hbm, init_hbm, ni_hbm, nj_hbm, y_out, h_out,
            p0, p1, ini, ni, nj, yb, hb):
    _, _, wid = _ids()
    base = wid * RPT_W * CC

    @pl.loop(0, NRCH)
    def _(t):
        e0 = base + t * CHW
        sl = pl.ds(e0, CHW)
        pltpu.sync_copy(part_hbm.at[sl], p0)
        pltpu.sync_copy(part_hbm.at[pl.ds(NP * CC + e0, CHW)], p1)
        pltpu.sync_copy(init_hbm.at[sl], ini)
        pltpu.sync_copy(ni_hbm.at[sl], ni)
        pltpu.sync_copy(nj_hbm.at[sl], nj)

        @pl.loop(0, RCH, unroll=4)
        def _(r):
            v = pl.ds(r * CC, CC)
            agg = p0[v] + p1[v]
            y = ini[v] + _f32(AA) * agg * ni[v]
            y = jnp.minimum(jnp.maximum(y, _f32(0.0)), _f32(1.0))
            yb[v] = y
            hb[v] = y * nj[v]

        pltpu.sync_copy(yb, y_out.at[sl])
        pltpu.sync_copy(hb, h_out.at[sl])


_BT = 2048


def _prep_body(lab_ref, msk_ref, dga_ref, dgb_ref,
               init_ref, h0_ref, ni_ref, nj_ref):
    lab = lab_ref[...]
    msk = msk_ref[...]
    deg = dga_ref[...] + dgb_ref[...]
    nrm = lax.rsqrt(jnp.maximum(deg, _f32(1.0)))
    ni = jnp.broadcast_to(nrm[:, 0:1], (_BT, CC))
    nj = jnp.broadcast_to(nrm[:, 8:9], (_BT, CC))
    y0 = jnp.where(msk > 0, lab, _f32(0.0))
    init_ref[...] = _f32(1.0 - AA) * y0
    ni_ref[...] = ni
    nj_ref[...] = nj
    h0_ref[...] = y0 * nj


def _prep(labels_p, mask_p, deg2):
    fspec = pl.BlockSpec((_BT, CC), lambda i: (i, 0))
    dspec_a = pl.BlockSpec((_BT, CC), lambda i: (i, 0))
    dspec_b = pl.BlockSpec((_BT, CC), lambda i: (i + NP // _BT, 0))
    return pl.pallas_call(
        _prep_body,
        grid=(NP // _BT,),
        in_specs=[fspec, pl.BlockSpec((_BT, 1), lambda i: (i, 0)),
                  dspec_a, dspec_b],
        out_specs=[fspec, fspec, fspec, fspec],
        out_shape=[jax.ShapeDtypeStruct((NP, CC), _f32)] * 4,
    )(labels_p, mask_p, deg2, deg2)


def kernel(labels, mask, edge_index):
    labels = labels.astype(_f32)
    src5 = edge_index[0].reshape(NW, NSUPER, SCH, 1, B)
    dst5 = edge_index[1].reshape(NW, NSUPER, SCH, 1, B)

    labels_p = jnp.pad(labels, ((0, NP - NN), (0, 0)))
    mask_p = jnp.pad(mask.astype(_i32), (0, NP - NN)).reshape(NP, 1)

    onesin = jnp.broadcast_to(
        jnp.array([1.0] * 8 + [0.0] * 8, _f32), (B, CC))
    onesout = jnp.broadcast_to(
        jnp.array([0.0] * 8 + [1.0] * 8, _f32), (B, CC))
    zerosC = jnp.zeros((RPT_SC, CC), _f32)

    deg2 = _degrees(src5, dst5, onesin, onesout, zerosC)
    init, h, ni, nj = _prep(labels_p, mask_p, deg2)
    init_f = init.reshape(NP * CC)
    ni_f = ni.reshape(NP * CC)
    nj_f = nj.reshape(NP * CC)

    y_f = None
    for _ in range(KK):
        part = _scatter(h, src5, dst5, zerosC)
        y_f, h_f = _update(part.reshape(2 * NP * CC), init_f, ni_f, nj_f)
        h = h_f.reshape(NP, CC)
    return y_f.reshape(NP, CC)[:NN]


# speedup vs baseline: 19.8670x; 1.0054x over previous
import functools

import jax
import jax.numpy as jnp
from jax import lax
from jax.experimental import pallas as pl
from jax.experimental.pallas import tpu as pltpu
from jax.experimental.pallas import tpu_sc as plsc

NN = 100000
NP = 102400
CC = 16
EE = 3200000
KK = 3
AA = 0.9

NC = 2
NS = 16
NW = NC * NS

B = 80
SCH = 10
EPW = EE // NW
NCHUNK = EPW // B
NSUPER = NCHUNK // SCH

RPT_SC = NP // NS
RPT_W = NP // NW
RCH = 128
CHW = RCH * CC
NRCH = RPT_W // RCH

_mesh = plsc.VectorSubcoreMesh(
    core_axis_name="c", subcore_axis_name="s", num_cores=NC, num_subcores=NS
)

_f32 = jnp.float32
_i32 = jnp.int32


def _ids():
    cid = lax.axis_index("c")
    sid = lax.axis_index("s")
    return cid, sid, sid * NC + cid


@functools.partial(
    pl.kernel,
    out_type=jax.ShapeDtypeStruct((2 * NP, CC), _f32),
    mesh=_mesh,
    compiler_params=pltpu.CompilerParams(use_tc_tiling_on_sc=False),
    scratch_types=(
        [pltpu.VMEM((SCH * B,), _i32), pltpu.VMEM((SCH * B,), _i32),
         pltpu.VMEM((B, CC), _f32), pltpu.VMEM((B, CC), _f32),
         pltpu.VMEM_SHARED((NP, CC), _f32),
         pltpu.SemaphoreType.DMA,
         pltpu.SemaphoreType.DMA,
         pltpu.SemaphoreType.DMA]
    ),
)
def _degrees(src5, dst5, onesin_hbm, onesout_hbm, zerosC_hbm, out, *scr):
    sidx, didx = scr[:2]
    onein, oneout, acc, isem, dsem, ssem = scr[2:]
    cid, sid, wid = _ids()
    pltpu.sync_copy(onesin_hbm, onein)
    pltpu.sync_copy(onesout_hbm, oneout)
    pltpu.sync_copy(zerosC_hbm, acc.at[pl.ds(sid * RPT_SC, RPT_SC)])
    plsc.subcore_barrier()

    @pl.loop(0, NSUPER)
    def _(g):
        p1 = pltpu.async_copy(src5.at[wid, g, 0], sidx, isem)
        p2 = pltpu.async_copy(dst5.at[wid, g, 0], didx, isem)
        p1.wait()
        p2.wait()
        dp = []
        for s in range(SCH):
            dp.append(pltpu.async_copy(onein, acc.at[didx.at[pl.ds(s * B, B)]],
                                       dsem, add=True))
            dp.append(pltpu.async_copy(oneout, acc.at[sidx.at[pl.ds(s * B, B)]],
                                       ssem, add=True))
        for p in dp:
            p.wait()

    plsc.subcore_barrier()
    sl = pl.ds(sid * RPT_SC, RPT_SC)
    pltpu.sync_copy(acc.at[sl], out.at[pl.ds(cid * NP + sid * RPT_SC, RPT_SC)])


@functools.partial(
    pl.kernel,
    out_type=jax.ShapeDtypeStruct((2 * NP, CC), _f32),
    mesh=_mesh,
    compiler_params=pltpu.CompilerParams(use_tc_tiling_on_sc=False),
    scratch_types=(
        [pltpu.VMEM((SCH * B,), _i32), pltpu.VMEM((SCH * B,), _i32),
         pltpu.VMEM((B, CC), _f32), pltpu.VMEM((B, CC), _f32),
         pltpu.VMEM_SHARED((NP, CC), _f32),
         pltpu.SemaphoreType.DMA,
         pltpu.SemaphoreType.DMA,
         pltpu.SemaphoreType.DMA,
         pltpu.SemaphoreType.DMA,
         pltpu.SemaphoreType.DMA]
    ),
)
def _scatter(h_hbm, src5, dst5, zeros_hbm, out, *scr):
    sidx, didx = scr[:2]
    rows = scr[2:4]
    acc = scr[4]
    isem, gsem0, gsem1, ssem0, ssem1 = scr[5:]
    gsems = (gsem0, gsem1)
    ssems = (ssem0, ssem1)
    cid, sid, wid = _ids()
    pltpu.sync_copy(zeros_hbm, acc.at[pl.ds(sid * RPT_SC, RPT_SC)])
    plsc.subcore_barrier()

    @pl.loop(0, NSUPER)
    def _(g):
        p1 = pltpu.async_copy(src5.at[wid, g, 0], sidx, isem)
        p2 = pltpu.async_copy(dst5.at[wid, g, 0], didx, isem)
        p1.wait()
        p2.wait()
        gp = [None, None]
        sp = [None, None]
        gp[0] = pltpu.async_copy(h_hbm.at[sidx.at[pl.ds(0, B)]], rows[0],
                                 gsems[0])
        for s in range(SCH):
            if s + 1 < SCH:
                j = (s + 1) % 2
                if sp[j] is not None:
                    sp[j].wait()
                    sp[j] = None
                gp[j] = pltpu.async_copy(
                    h_hbm.at[sidx.at[pl.ds((s + 1) * B, B)]], rows[j], gsems[j])
            gp[s % 2].wait()
            sp[s % 2] = pltpu.async_copy(rows[s % 2],
                                         acc.at[didx.at[pl.ds(s * B, B)]],
                                         ssems[s % 2], add=True)
        for p in sp:
            if p is not None:
                p.wait()

    plsc.subcore_barrier()
    sl = pl.ds(sid * RPT_SC, RPT_SC)
    pltpu.sync_copy(acc.at[sl], out.at[pl.ds(cid * NP + sid * RPT_SC, RPT_SC)])


@functools.partial(
    pl.kernel,
    out_type=(jax.ShapeDtypeStruct((NP * CC,), _f32),
              jax.ShapeDtypeStruct((NP * CC,), _f32)),
    mesh=_mesh,
    compiler_params=pltpu.CompilerParams(use_tc_tiling_on_sc=False),
    scratch_types=[pltpu.VMEM((CHW,), _f32)] * 7,
)
def _update(part_hbm, init_hbm, ni_hbm, nj_hbm, y_out, h_out,
            p0, p1, ini, ni, nj, yb, hb):
    _, _, wid = _ids()
    base = wid * RPT_W * CC

    @pl.loop(0, NRCH)
    def _(t):
        e0 = base + t * CHW
        sl = pl.ds(e0, CHW)
        pltpu.sync_copy(part_hbm.at[sl], p0)
        pltpu.sync_copy(part_hbm.at[pl.ds(NP * CC + e0, CHW)], p1)
        pltpu.sync_copy(init_hbm.at[sl], ini)
        pltpu.sync_copy(ni_hbm.at[sl], ni)
        pltpu.sync_copy(nj_hbm.at[sl], nj)

        @pl.loop(0, RCH, unroll=4)
        def _(r):
            v = pl.ds(r * CC, CC)
            agg = p0[v] + p1[v]
            y = ini[v] + _f32(AA) * agg * ni[v]
            y = jnp.minimum(jnp.maximum(y, _f32(0.0)), _f32(1.0))
            yb[v] = y
            hb[v] = y * nj[v]

        pltpu.sync_copy(yb, y_out.at[sl])
        pltpu.sync_copy(hb, h_out.at[sl])


_BT = 2048


def _prep_body(lab_ref, msk_ref, dga_ref, dgb_ref,
               init_ref, h0_ref, ni_ref, nj_ref):
    lab = lab_ref[...]
    msk = msk_ref[...]
    deg = dga_ref[...] + dgb_ref[...]
    nrm = lax.rsqrt(jnp.maximum(deg, _f32(1.0)))
    ni = jnp.broadcast_to(nrm[:, 0:1], (_BT, CC))
    nj = jnp.broadcast_to(nrm[:, 8:9], (_BT, CC))
    y0 = jnp.where(msk > 0, lab, _f32(0.0))
    init_ref[...] = _f32(1.0 - AA) * y0
    ni_ref[...] = ni
    nj_ref[...] = nj
    h0_ref[...] = y0 * nj


def _prep(labels_p, mask_p, deg2):
    fspec = pl.BlockSpec((_BT, CC), lambda i: (i, 0))
    dspec_a = pl.BlockSpec((_BT, CC), lambda i: (i, 0))
    dspec_b = pl.BlockSpec((_BT, CC), lambda i: (i + NP // _BT, 0))
    return pl.pallas_call(
        _prep_body,
        grid=(NP // _BT,),
        in_specs=[fspec, pl.BlockSpec((_BT, 1), lambda i: (i, 0)),
                  dspec_a, dspec_b],
        out_specs=[fspec, fspec, fspec, fspec],
        out_shape=[jax.ShapeDtypeStruct((NP, CC), _f32)] * 4,
    )(labels_p, mask_p, deg2, deg2)


def kernel(labels, mask, edge_index):
    labels = labels.astype(_f32)
    src5 = edge_index[0].reshape(NW, NSUPER, 1, SCH * B)
    dst5 = edge_index[1].reshape(NW, NSUPER, 1, SCH * B)

    labels_p = jnp.pad(labels, ((0, NP - NN), (0, 0)))
    mask_p = jnp.pad(mask.astype(_i32), (0, NP - NN)).reshape(NP, 1)

    onesin = jnp.broadcast_to(
        jnp.array([1.0] * 8 + [0.0] * 8, _f32), (B, CC))
    onesout = jnp.broadcast_to(
        jnp.array([0.0] * 8 + [1.0] * 8, _f32), (B, CC))
    zerosC = jnp.zeros((RPT_SC, CC), _f32)

    deg2 = _degrees(src5, dst5, onesin, onesout, zerosC)
    init, h, ni, nj = _prep(labels_p, mask_p, deg2)
    init_f = init.reshape(NP * CC)
    ni_f = ni.reshape(NP * CC)
    nj_f = nj.reshape(NP * CC)

    y_f = None
    for _ in range(KK):
        part = _scatter(h, src5, dst5, zerosC)
        y_f, h_f = _update(part.reshape(2 * NP * CC), init_f, ni_f, nj_f)
        h = h_f.reshape(NP, CC)
    return y_f.reshape(NP, CC)[:NN]


# speedup vs baseline: 28.1607x; 1.4175x over previous
import functools

import jax
import jax.numpy as jnp
from jax import lax
from jax.experimental import pallas as pl
from jax.experimental.pallas import tpu as pltpu
from jax.experimental.pallas import tpu_sc as plsc

NN = 100000
NP = 102400
CC = 16
EE = 3200000
KK = 3
AA = 0.9

NC = 2
NS = 16
NW = NC * NS

B = 80
SCH = 25
EPW = EE // NW
NCHUNK = EPW // B
NSUPER = NCHUNK // SCH
RING = 4

RPT_SC = NP // NS
RPT_W = NP // NW
RCH = 128
CHW = RCH * CC
NRCH = RPT_W // RCH

_mesh = plsc.VectorSubcoreMesh(
    core_axis_name="c", subcore_axis_name="s", num_cores=NC, num_subcores=NS
)

_f32 = jnp.float32
_i32 = jnp.int32


def _ids():
    cid = lax.axis_index("c")
    sid = lax.axis_index("s")
    return cid, sid, sid * NC + cid


@functools.partial(
    pl.kernel,
    out_type=jax.ShapeDtypeStruct((2 * NP, CC), _f32),
    mesh=_mesh,
    compiler_params=pltpu.CompilerParams(use_tc_tiling_on_sc=False),
    scratch_types=(
        [pltpu.VMEM((SCH * B,), _i32), pltpu.VMEM((SCH * B,), _i32),
         pltpu.VMEM((B, CC), _f32), pltpu.VMEM((B, CC), _f32),
         pltpu.VMEM_SHARED((NP, CC), _f32),
         pltpu.SemaphoreType.DMA,
         pltpu.SemaphoreType.DMA,
         pltpu.SemaphoreType.DMA]
    ),
)
def _degrees(src5, dst5, onesin_hbm, onesout_hbm, zerosC_hbm, out, *scr):
    sidx, didx = scr[:2]
    onein, oneout, acc, isem, dsem, ssem = scr[2:]
    cid, sid, wid = _ids()
    pltpu.sync_copy(onesin_hbm, onein)
    pltpu.sync_copy(onesout_hbm, oneout)
    pltpu.sync_copy(zerosC_hbm, acc.at[pl.ds(sid * RPT_SC, RPT_SC)])
    plsc.subcore_barrier()

    @pl.loop(0, NSUPER)
    def _(g):
        p1 = pltpu.async_copy(src5.at[wid, g, 0], sidx, isem)
        p2 = pltpu.async_copy(dst5.at[wid, g, 0], didx, isem)
        p1.wait()
        p2.wait()
        dp = []
        for s in range(SCH):
            dp.append(pltpu.async_copy(onein, acc.at[didx.at[pl.ds(s * B, B)]],
                                       dsem, add=True))
            dp.append(pltpu.async_copy(oneout, acc.at[sidx.at[pl.ds(s * B, B)]],
                                       ssem, add=True))
        for p in dp:
            p.wait()

    plsc.subcore_barrier()
    sl = pl.ds(sid * RPT_SC, RPT_SC)
    pltpu.sync_copy(acc.at[sl], out.at[pl.ds(cid * NP + sid * RPT_SC, RPT_SC)])


@functools.partial(
    pl.kernel,
    out_type=jax.ShapeDtypeStruct((2 * NP, CC), _f32),
    mesh=_mesh,
    compiler_params=pltpu.CompilerParams(use_tc_tiling_on_sc=False),
    scratch_types=(
        [pltpu.VMEM((SCH * B,), _i32)] * 4
        + [pltpu.VMEM((B, CC), _f32)] * RING
        + [pltpu.VMEM_SHARED((NP, CC), _f32)]
        + [pltpu.SemaphoreType.DMA] * (2 + 2 * RING)
    ),
)
def _scatter(h_hbm, src5, dst5, zeros_hbm, out, *scr):
    sidx = scr[0:2]
    didx = scr[2:4]
    rows = scr[4:4 + RING]
    acc = scr[4 + RING]
    isems = scr[5 + RING:7 + RING]
    gsems = scr[7 + RING:7 + 2 * RING]
    ssems = scr[7 + 2 * RING:7 + 3 * RING]
    cid, sid, wid = _ids()
    pltpu.sync_copy(zeros_hbm, acc.at[pl.ds(sid * RPT_SC, RPT_SC)])
    plsc.subcore_barrier()

    pltpu.async_copy(src5.at[wid, 0, 0], sidx[0], isems[0]).wait()
    pltpu.async_copy(dst5.at[wid, 0, 0], didx[0], isems[0]).wait()

    def run_super(g, cur, nxt):
        si, di = sidx[cur], didx[cur]
        gn = jnp.minimum(g + 1, NSUPER - 1)
        ip = [pltpu.async_copy(src5.at[wid, gn, 0], sidx[nxt], isems[nxt]),
              pltpu.async_copy(dst5.at[wid, gn, 0], didx[nxt], isems[nxt])]
        gp = [None] * RING
        sp = [None] * RING
        for j in range(RING - 1):
            gp[j] = pltpu.async_copy(h_hbm.at[si.at[pl.ds(j * B, B)]],
                                     rows[j], gsems[j])
        for s in range(SCH):
            j = s % RING
            gp[j].wait()
            sp[j] = pltpu.async_copy(rows[j], acc.at[di.at[pl.ds(s * B, B)]],
                                     ssems[j], add=True)
            nx = s + RING - 1
            if nx < SCH:
                jn = nx % RING
                if sp[jn] is not None:
                    sp[jn].wait()
                    sp[jn] = None
                gp[jn] = pltpu.async_copy(h_hbm.at[si.at[pl.ds(nx * B, B)]],
                                          rows[jn], gsems[jn])
        for p in sp:
            if p is not None:
                p.wait()
        return ip

    @pl.loop(0, NSUPER // 2)
    def _(h):
        g0 = 2 * h
        for p in run_super(g0, 0, 1):
            p.wait()
        for p in run_super(g0 + 1, 1, 0):
            p.wait()

    plsc.subcore_barrier()
    sl = pl.ds(sid * RPT_SC, RPT_SC)
    pltpu.sync_copy(acc.at[sl], out.at[pl.ds(cid * NP + sid * RPT_SC, RPT_SC)])


@functools.partial(
    pl.kernel,
    out_type=(jax.ShapeDtypeStruct((NP * CC,), _f32),
              jax.ShapeDtypeStruct((NP * CC,), _f32)),
    mesh=_mesh,
    compiler_params=pltpu.CompilerParams(use_tc_tiling_on_sc=False),
    scratch_types=[pltpu.VMEM((CHW,), _f32)] * 7,
)
def _update(part_hbm, init_hbm, ni_hbm, nj_hbm, y_out, h_out,
            p0, p1, ini, ni, nj, yb, hb):
    _, _, wid = _ids()
    base = wid * RPT_W * CC

    @pl.loop(0, NRCH)
    def _(t):
        e0 = base + t * CHW
        sl = pl.ds(e0, CHW)
        pltpu.sync_copy(part_hbm.at[sl], p0)
        pltpu.sync_copy(part_hbm.at[pl.ds(NP * CC + e0, CHW)], p1)
        pltpu.sync_copy(init_hbm.at[sl], ini)
        pltpu.sync_copy(ni_hbm.at[sl], ni)
        pltpu.sync_copy(nj_hbm.at[sl], nj)

        @pl.loop(0, RCH, unroll=4)
        def _(r):
            v = pl.ds(r * CC, CC)
            agg = p0[v] + p1[v]
            y = ini[v] + _f32(AA) * agg * ni[v]
            y = jnp.minimum(jnp.maximum(y, _f32(0.0)), _f32(1.0))
            yb[v] = y
            hb[v] = y * nj[v]

        pltpu.sync_copy(yb, y_out.at[sl])
        pltpu.sync_copy(hb, h_out.at[sl])


_BT = 2048


def _prep_body(lab_ref, msk_ref, dga_ref, dgb_ref,
               init_ref, h0_ref, ni_ref, nj_ref):
    lab = lab_ref[...]
    msk = msk_ref[...]
    deg = dga_ref[...] + dgb_ref[...]
    nrm = lax.rsqrt(jnp.maximum(deg, _f32(1.0)))
    ni = jnp.broadcast_to(nrm[:, 0:1], (_BT, CC))
    nj = jnp.broadcast_to(nrm[:, 8:9], (_BT, CC))
    y0 = jnp.where(msk > 0, lab, _f32(0.0))
    init_ref[...] = _f32(1.0 - AA) * y0
    ni_ref[...] = ni
    nj_ref[...] = nj
    h0_ref[...] = y0 * nj


def _prep(labels_p, mask_p, deg2):
    fspec = pl.BlockSpec((_BT, CC), lambda i: (i, 0))
    dspec_a = pl.BlockSpec((_BT, CC), lambda i: (i, 0))
    dspec_b = pl.BlockSpec((_BT, CC), lambda i: (i + NP // _BT, 0))
    return pl.pallas_call(
        _prep_body,
        grid=(NP // _BT,),
        in_specs=[fspec, pl.BlockSpec((_BT, 1), lambda i: (i, 0)),
                  dspec_a, dspec_b],
        out_specs=[fspec, fspec, fspec, fspec],
        out_shape=[jax.ShapeDtypeStruct((NP, CC), _f32)] * 4,
    )(labels_p, mask_p, deg2, deg2)


def kernel(labels, mask, edge_index):
    labels = labels.astype(_f32)
    src5 = edge_index[0].reshape(NW, NSUPER, 1, SCH * B)
    dst5 = edge_index[1].reshape(NW, NSUPER, 1, SCH * B)

    labels_p = jnp.pad(labels, ((0, NP - NN), (0, 0)))
    mask_p = jnp.pad(mask.astype(_i32), (0, NP - NN)).reshape(NP, 1)

    onesin = jnp.broadcast_to(
        jnp.array([1.0] * 8 + [0.0] * 8, _f32), (B, CC))
    onesout = jnp.broadcast_to(
        jnp.array([0.0] * 8 + [1.0] * 8, _f32), (B, CC))
    zerosC = jnp.zeros((RPT_SC, CC), _f32)

    deg2 = _degrees(src5, dst5, onesin, onesout, zerosC)
    init, h, ni, nj = _prep(labels_p, mask_p, deg2)
    init_f = init.reshape(NP * CC)
    ni_f = ni.reshape(NP * CC)
    nj_f = nj.reshape(NP * CC)

    y_f = None
    for _ in range(KK):
        part = _scatter(h, src5, dst5, zerosC)
        y_f, h_f = _update(part.reshape(2 * NP * CC), init_f, ni_f, nj_f)
        h = h_f.reshape(NP, CC)
    return y_f.reshape(NP, CC)[:NN]


# speedup vs baseline: 31.1642x; 1.1067x over previous
import functools

import jax
import jax.numpy as jnp
from jax import lax
from jax.experimental import pallas as pl
from jax.experimental.pallas import tpu as pltpu
from jax.experimental.pallas import tpu_sc as plsc

NN = 100000
NP = 102400
CC = 16
EE = 3200000
KK = 3
AA = 0.9

NC = 2
NS = 16
NW = NC * NS

B = 80
SCH = 25
EPW = EE // NW
NCHUNK = EPW // B
NSUPER = NCHUNK // SCH
RING = 4

RPT_SC = NP // NS
RPT_W = NP // NW
RCH = 40
CHW = RCH * CC
NRCH = RPT_W // RCH
SLOTS = 4

_mesh = plsc.VectorSubcoreMesh(
    core_axis_name="c", subcore_axis_name="s", num_cores=NC, num_subcores=NS
)

_f32 = jnp.float32
_i32 = jnp.int32


def _ids():
    cid = lax.axis_index("c")
    sid = lax.axis_index("s")
    return cid, sid, sid * NC + cid


@functools.partial(
    pl.kernel,
    out_type=jax.ShapeDtypeStruct((2 * NP, CC), _f32),
    mesh=_mesh,
    compiler_params=pltpu.CompilerParams(use_tc_tiling_on_sc=False),
    scratch_types=(
        [pltpu.VMEM((SCH * B,), _i32), pltpu.VMEM((SCH * B,), _i32),
         pltpu.VMEM((B, CC), _f32), pltpu.VMEM((B, CC), _f32),
         pltpu.VMEM_SHARED((NP, CC), _f32),
         pltpu.SemaphoreType.DMA,
         pltpu.SemaphoreType.DMA,
         pltpu.SemaphoreType.DMA]
    ),
)
def _degrees(src5, dst5, onesin_hbm, onesout_hbm, zerosC_hbm, out, *scr):
    sidx, didx = scr[:2]
    onein, oneout, acc, isem, dsem, ssem = scr[2:]
    cid, sid, wid = _ids()
    pltpu.sync_copy(onesin_hbm, onein)
    pltpu.sync_copy(onesout_hbm, oneout)
    pltpu.sync_copy(zerosC_hbm, acc.at[pl.ds(sid * RPT_SC, RPT_SC)])
    plsc.subcore_barrier()

    @pl.loop(0, NSUPER)
    def _(g):
        p1 = pltpu.async_copy(src5.at[wid, g, 0], sidx, isem)
        p2 = pltpu.async_copy(dst5.at[wid, g, 0], didx, isem)
        p1.wait()
        p2.wait()
        dp = []
        for s in range(SCH):
            dp.append(pltpu.async_copy(onein, acc.at[didx.at[pl.ds(s * B, B)]],
                                       dsem, add=True))
            dp.append(pltpu.async_copy(oneout, acc.at[sidx.at[pl.ds(s * B, B)]],
                                       ssem, add=True))
        for p in dp:
            p.wait()

    plsc.subcore_barrier()
    sl = pl.ds(sid * RPT_SC, RPT_SC)
    pltpu.sync_copy(acc.at[sl], out.at[pl.ds(cid * NP + sid * RPT_SC, RPT_SC)])


@functools.partial(
    pl.kernel,
    out_type=jax.ShapeDtypeStruct((2 * NP, CC), _f32),
    mesh=_mesh,
    compiler_params=pltpu.CompilerParams(use_tc_tiling_on_sc=False),
    scratch_types=(
        [pltpu.VMEM((SCH * B,), _i32)] * 4
        + [pltpu.VMEM((B, CC), _f32)] * RING
        + [pltpu.VMEM_SHARED((NP, CC), _f32)]
        + [pltpu.SemaphoreType.DMA] * (2 + 2 * RING)
    ),
)
def _scatter(h_hbm, src5, dst5, zeros_hbm, out, *scr):
    sidx = scr[0:2]
    didx = scr[2:4]
    rows = scr[4:4 + RING]
    acc = scr[4 + RING]
    isems = scr[5 + RING:7 + RING]
    gsems = scr[7 + RING:7 + 2 * RING]
    ssems = scr[7 + 2 * RING:7 + 3 * RING]
    cid, sid, wid = _ids()
    pltpu.sync_copy(zeros_hbm, acc.at[pl.ds(sid * RPT_SC, RPT_SC)])
    plsc.subcore_barrier()

    pltpu.async_copy(src5.at[wid, 0, 0], sidx[0], isems[0]).wait()
    pltpu.async_copy(dst5.at[wid, 0, 0], didx[0], isems[0]).wait()

    def run_super(g, cur, nxt):
        si, di = sidx[cur], didx[cur]
        gn = jnp.minimum(g + 1, NSUPER - 1)
        ip = [pltpu.async_copy(src5.at[wid, gn, 0], sidx[nxt], isems[nxt]),
              pltpu.async_copy(dst5.at[wid, gn, 0], didx[nxt], isems[nxt])]
        gp = [None] * RING
        sp = [None] * RING
        for j in range(RING - 1):
            gp[j] = pltpu.async_copy(h_hbm.at[si.at[pl.ds(j * B, B)]],
                                     rows[j], gsems[j])
        for s in range(SCH):
            j = s % RING
            gp[j].wait()
            sp[j] = pltpu.async_copy(rows[j], acc.at[di.at[pl.ds(s * B, B)]],
                                     ssems[j], add=True)
            nx = s + RING - 1
            if nx < SCH:
                jn = nx % RING
                if sp[jn] is not None:
                    sp[jn].wait()
                    sp[jn] = None
                gp[jn] = pltpu.async_copy(h_hbm.at[si.at[pl.ds(nx * B, B)]],
                                          rows[jn], gsems[jn])
        for p in sp:
            if p is not None:
                p.wait()
        return ip

    @pl.loop(0, NSUPER // 2)
    def _(h):
        g0 = 2 * h
        for p in run_super(g0, 0, 1):
            p.wait()
        for p in run_super(g0 + 1, 1, 0):
            p.wait()

    plsc.subcore_barrier()
    sl = pl.ds(sid * RPT_SC, RPT_SC)
    pltpu.sync_copy(acc.at[sl], out.at[pl.ds(cid * NP + sid * RPT_SC, RPT_SC)])


@functools.partial(
    pl.kernel,
    out_type=(jax.ShapeDtypeStruct((NP * CC,), _f32),
              jax.ShapeDtypeStruct((NP * CC,), _f32)),
    mesh=_mesh,
    compiler_params=pltpu.CompilerParams(use_tc_tiling_on_sc=False),
    scratch_types=(
        [pltpu.VMEM((CHW,), _f32)] * (7 * SLOTS)
        + [pltpu.SemaphoreType.DMA] * (2 * SLOTS)
    ),
)
def _update(part_hbm, init_hbm, ni_hbm, nj_hbm, y_out, h_out, *scr):
    p0 = scr[0:SLOTS]
    p1 = scr[SLOTS:2 * SLOTS]
    ini = scr[2 * SLOTS:3 * SLOTS]
    ni = scr[3 * SLOTS:4 * SLOTS]
    nj = scr[4 * SLOTS:5 * SLOTS]
    yb = scr[5 * SLOTS:6 * SLOTS]
    hb = scr[6 * SLOTS:7 * SLOTS]
    lsems = scr[7 * SLOTS:8 * SLOTS]
    ssems = scr[8 * SLOTS:9 * SLOTS]
    _, _, wid = _ids()
    base = wid * RPT_W * CC

    @pl.loop(0, NRCH // SLOTS)
    def _(u):
        t0 = base + u * SLOTS * CHW
        lps = []
        for j in range(SLOTS):
            e0 = t0 + j * CHW
            sl = pl.ds(e0, CHW)
            lps.append([
                pltpu.async_copy(part_hbm.at[sl], p0[j], lsems[j]),
                pltpu.async_copy(part_hbm.at[pl.ds(NP * CC + e0, CHW)],
                                 p1[j], lsems[j]),
                pltpu.async_copy(init_hbm.at[sl], ini[j], lsems[j]),
                pltpu.async_copy(ni_hbm.at[sl], ni[j], lsems[j]),
                pltpu.async_copy(nj_hbm.at[sl], nj[j], lsems[j]),
            ])
        sps = []
        for j in range(SLOTS):
            for p in lps[j]:
                p.wait()

            @pl.loop(0, RCH, unroll=4)
            def _(r, j=j):
                v = pl.ds(r * CC, CC)
                agg = p0[j][v] + p1[j][v]
                y = ini[j][v] + _f32(AA) * agg * ni[j][v]
                y = jnp.minimum(jnp.maximum(y, _f32(0.0)), _f32(1.0))
                yb[j][v] = y
                hb[j][v] = y * nj[j][v]

            sl = pl.ds(t0 + j * CHW, CHW)
            sps.append(pltpu.async_copy(yb[j], y_out.at[sl], ssems[j]))
            sps.append(pltpu.async_copy(hb[j], h_out.at[sl], ssems[j]))
        for p in sps:
            p.wait()


_BT = 2048


def _prep_body(lab_ref, msk_ref, dga_ref, dgb_ref,
               init_ref, h0_ref, ni_ref, nj_ref):
    lab = lab_ref[...]
    msk = msk_ref[...]
    deg = dga_ref[...] + dgb_ref[...]
    nrm = lax.rsqrt(jnp.maximum(deg, _f32(1.0)))
    ni = jnp.broadcast_to(nrm[:, 0:1], (_BT, CC))
    nj = jnp.broadcast_to(nrm[:, 8:9], (_BT, CC))
    y0 = jnp.where(msk > 0, lab, _f32(0.0))
    init_ref[...] = _f32(1.0 - AA) * y0
    ni_ref[...] = ni
    nj_ref[...] = nj
    h0_ref[...] = y0 * nj


def _prep(labels_p, mask_p, deg2):
    fspec = pl.BlockSpec((_BT, CC), lambda i: (i, 0))
    dspec_a = pl.BlockSpec((_BT, CC), lambda i: (i, 0))
    dspec_b = pl.BlockSpec((_BT, CC), lambda i: (i + NP // _BT, 0))
    return pl.pallas_call(
        _prep_body,
        grid=(NP // _BT,),
        in_specs=[fspec, pl.BlockSpec((_BT, 1), lambda i: (i, 0)),
                  dspec_a, dspec_b],
        out_specs=[fspec, fspec, fspec, fspec],
        out_shape=[jax.ShapeDtypeStruct((NP, CC), _f32)] * 4,
    )(labels_p, mask_p, deg2, deg2)


def kernel(labels, mask, edge_index):
    labels = labels.astype(_f32)
    src5 = edge_index[0].reshape(NW, NSUPER, 1, SCH * B)
    dst5 = edge_index[1].reshape(NW, NSUPER, 1, SCH * B)

    labels_p = jnp.pad(labels, ((0, NP - NN), (0, 0)))
    mask_p = jnp.pad(mask.astype(_i32), (0, NP - NN)).reshape(NP, 1)

    onesin = jnp.broadcast_to(
        jnp.array([1.0] * 8 + [0.0] * 8, _f32), (B, CC))
    onesout = jnp.broadcast_to(
        jnp.array([0.0] * 8 + [1.0] * 8, _f32), (B, CC))
    zerosC = jnp.zeros((RPT_SC, CC), _f32)

    deg2 = _degrees(src5, dst5, onesin, onesout, zerosC)
    init, h, ni, nj = _prep(labels_p, mask_p, deg2)
    init_f = init.reshape(NP * CC)
    ni_f = ni.reshape(NP * CC)
    nj_f = nj.reshape(NP * CC)

    y_f = None
    for _ in range(KK):
        part = _scatter(h, src5, dst5, zerosC)
        y_f, h_f = _update(part.reshape(2 * NP * CC), init_f, ni_f, nj_f)
        h = h_f.reshape(NP, CC)
    return y_f.reshape(NP, CC)[:NN]
